# Initial kernel scaffold; baseline (speedup 1.0000x reference)
import jax
import jax.numpy as jnp
from jax.experimental import pallas as pl


def kernel(x, token_embedding_weight):
    raise NotImplementedError("write your pallas kernel here")


# speedup vs baseline: 1.4395x; 1.4395x over previous
import functools

import jax
import jax.numpy as jnp
from jax import lax
from jax.experimental import pallas as pl
from jax.experimental.pallas import tpu as pltpu
from jax.experimental.pallas import tpu_sc as plsc

_VOCAB = 1000
_BATCH = 4096
_HIST = 20
_D = _VOCAB
_B = _BATCH * _HIST

_NC = 2
_NS = 16
_NW = _NC * _NS
_BPW = _B // _NW
_C = 40
_NCHUNK = _BPW // _C

_mesh = plsc.VectorSubcoreMesh(
    core_axis_name="c", subcore_axis_name="s", num_cores=_NC, num_subcores=_NS
)


@functools.partial(
    pl.kernel,
    out_type=jax.ShapeDtypeStruct((_B, _D), jnp.float32),
    mesh=_mesh,
    scratch_types=[
        pltpu.VMEM((_BPW,), jnp.int32),
        pltpu.VMEM((_C, _D), jnp.float32),
        pltpu.VMEM((_C, _D), jnp.float32),
        pltpu.SemaphoreType.DMA,
        pltpu.SemaphoreType.DMA,
    ],
    compiler_params=pltpu.CompilerParams(use_tc_tiling_on_sc=False),
)
def _embed_lookup(idx_hbm, table_hbm, out_hbm, idx_v, buf0, buf1, sem0, sem1):
    wid = lax.axis_index("s") * _NC + lax.axis_index("c")
    base = wid * _BPW
    pltpu.sync_copy(idx_hbm.at[pl.ds(base, _BPW)], idx_v)

    def gather_start(chunk, buf, sem):
        idx_slice = idx_v.at[pl.ds(chunk * _C, _C)]
        pltpu.async_copy(table_hbm.at[idx_slice], buf, sem)

    def gather_wait(chunk, buf, sem):
        idx_slice = idx_v.at[pl.ds(chunk * _C, _C)]
        pltpu.make_async_copy(table_hbm.at[idx_slice], buf, sem).wait()

    def write_out(chunk, buf):
        pltpu.sync_copy(buf, out_hbm.at[pl.ds(base + chunk * _C, _C)])

    gather_start(0, buf0, sem0)
    gather_start(1, buf1, sem1)

    def body(t, carry):
        c0 = 2 * t
        gather_wait(c0, buf0, sem0)
        write_out(c0, buf0)
        gather_start(c0 + 2, buf0, sem0)
        gather_wait(c0 + 1, buf1, sem1)
        write_out(c0 + 1, buf1)
        gather_start(c0 + 3, buf1, sem1)
        return carry

    lax.fori_loop(0, _NCHUNK // 2 - 1, body, 0)

    gather_wait(_NCHUNK - 2, buf0, sem0)
    write_out(_NCHUNK - 2, buf0)
    gather_wait(_NCHUNK - 1, buf1, sem1)
    write_out(_NCHUNK - 1, buf1)


def kernel(x, token_embedding_weight):
    idx = x.reshape(-1).astype(jnp.int32)
    out = _embed_lookup(idx, token_embedding_weight)
    return out.reshape(_BATCH, _HIST, _VOCAB)
